# baseline (speedup 1.0000x reference)
import jax
import jax.numpy as jnp
from jax.experimental import pallas as pl
from jax.experimental.pallas import tpu as pltpu

_LANE = 128
_C = 4096
_LOG2E = 1.4426950408889634
_LOG2_EPS = -53.150849518197795


def _fold(x, width):
    parts = [x[:, c:c + width] for c in range(0, x.shape[1], width)]
    while len(parts) > 1:
        parts = [parts[j] + parts[j + 1] for j in range(0, len(parts) - 1, 2)] \
            + ([parts[-1]] if len(parts) % 2 else [])
    return parts[0]


def _ntm_block(k_ref, beta_ref, g_ref, s_ref, gamma_ref, wp_ref, memT_ref,
               out_ref, mhl_ref, ep_ref):
    M = memT_ref.shape[0]
    N = wp_ref.shape[1]
    H = _LANE

    @pl.when(pl.program_id(0) == 0)
    def _():
        mt = memT_ref[...]
        mn = mt * jax.lax.rsqrt(
            jnp.sum(mt * mt, axis=0, keepdims=True) + 1e-30)
        mh = mn.astype(jnp.bfloat16)
        ml = (mn - mh.astype(jnp.float32)).astype(jnp.bfloat16)
        mhl_ref[0:M, :] = mh
        mhl_ref[M:2 * M, :] = ml
        mhl_ref[2 * M:3 * M, :] = mh

    beta = jax.nn.softplus(beta_ref[...])
    g = jax.nn.sigmoid(g_ref[...])
    s = jax.nn.softmax(s_ref[...], axis=1)
    gamma = 1.0 + jax.nn.softplus(gamma_ref[...])

    kb = k_ref[...]
    k2 = jnp.sum(kb * kb, axis=1, keepdims=True)
    kbs = kb * (beta * (_LOG2E * jax.lax.rsqrt(k2 + 1e-30)))
    kh = kbs.astype(jnp.bfloat16)
    kl = (kbs - kh.astype(jnp.float32)).astype(jnp.bfloat16)
    a = jnp.concatenate([kh, kh, kl], axis=1)

    eacc = None
    for base in range(0, N, _C):
        e_c = jnp.exp2(jax.lax.dot_general(
            a, mhl_ref[:, base:base + _C], (((1,), (0,)), ((), ())),
            preferred_element_type=jnp.float32))
        ep_ref[:, H + base:H + base + _C] = e_c
        if base == 0:
            ep_ref[:, N + H:N + 2 * H] = e_c[:, 0:H]
        if base == N - _C:
            ep_ref[:, 0:H] = e_c[:, _C - H:_C]
        part = _fold(e_c, H)
        eacc = part if eacc is None else eacc + part
    esum = jnp.sum(eacc, axis=1, keepdims=True)

    ga = g / esum
    gb = 1.0 - g
    s1 = s[:, 1:2]
    r0 = s[:, 0:1] / s1
    r2 = s[:, 2:3] / s1
    eps = jnp.exp2(gamma * (-jnp.log2(s1)) + _LOG2_EPS)

    acc = None
    for base in range(0, N, _C):
        eext = ep_ref[:, base:base + _C + 2 * H]
        if base == 0:
            wpext = jnp.concatenate(
                [wp_ref[:, N - H:N], wp_ref[:, 0:_C + H]], axis=1)
        elif base == N - _C:
            wpext = jnp.concatenate(
                [wp_ref[:, base - H:N], wp_ref[:, 0:H]], axis=1)
        else:
            wpext = wp_ref[:, base - H:base + _C + H]
        wgext = eext * ga + wpext * gb
        wl = wgext[:, H - 1:_C + H - 1]
        wr = wgext[:, H + 1:_C + H + 1]
        wh = wgext[:, H:_C + H] + wl * r0 + wr * r2
        wc = jnp.exp2(gamma * jnp.log2(wh))
        out_ref[:, base:base + _C] = wc
        part = _fold(wc, H)
        acc = part if acc is None else acc + part

    total = jnp.sum(acc, axis=1, keepdims=True) + eps
    out_ref[...] = out_ref[...] * (1.0 / total)


@jax.jit
def kernel(k, beta, g, s, gamma, w_prev, memory):
    B, M = k.shape
    N = memory.shape[0]
    BB = 64
    memT = memory.T

    grid = (B // BB,)
    return pl.pallas_call(
        _ntm_block,
        grid=grid,
        in_specs=[
            pl.BlockSpec((BB, M), lambda i: (i, 0)),
            pl.BlockSpec((BB, 1), lambda i: (i, 0)),
            pl.BlockSpec((BB, 1), lambda i: (i, 0)),
            pl.BlockSpec((BB, 3), lambda i: (i, 0)),
            pl.BlockSpec((BB, 1), lambda i: (i, 0)),
            pl.BlockSpec((BB, N), lambda i: (i, 0)),
            pl.BlockSpec((M, N), lambda i: (0, 0)),
        ],
        out_specs=pl.BlockSpec((BB, N), lambda i: (i, 0)),
        out_shape=jax.ShapeDtypeStruct((B, N), jnp.float32),
        scratch_shapes=[
            pltpu.VMEM((3 * M, N), jnp.bfloat16),
            pltpu.VMEM((BB, N + 2 * _LANE), jnp.float32),
        ],
    )(k, beta, g, s, gamma, w_prev, memT)


# speedup vs baseline: 1.0269x; 1.0269x over previous
import jax
import jax.numpy as jnp
from jax.experimental import pallas as pl
from jax.experimental.pallas import tpu as pltpu

_LANE = 128
_C = 4096
_LOG2E = 1.4426950408889634
_LOG2_EPS = -53.150849518197795


def _ntm_block(k_ref, beta_ref, g_ref, s_ref, gamma_ref, wp_ref, memT_ref,
               out_ref, mhl_ref, ep_ref):
    M = memT_ref.shape[0]
    N = wp_ref.shape[1]
    H = _LANE

    @pl.when(pl.program_id(0) == 0)
    def _():
        mt = memT_ref[...]
        mn = mt * jax.lax.rsqrt(
            jnp.sum(mt * mt, axis=0, keepdims=True) + 1e-30)
        mh = mn.astype(jnp.bfloat16)
        ml = (mn - mh.astype(jnp.float32)).astype(jnp.bfloat16)
        mhl_ref[0:M, :] = mh
        mhl_ref[M:2 * M, :] = ml
        mhl_ref[2 * M:3 * M, :] = mh

    beta = jax.nn.softplus(beta_ref[...])
    g = jax.nn.sigmoid(g_ref[...])
    s = jax.nn.softmax(s_ref[...], axis=1)
    gamma = 1.0 + jax.nn.softplus(gamma_ref[...])

    kb = k_ref[...]
    k2 = jnp.sum(kb * kb, axis=1, keepdims=True)
    kbs = kb * (beta * (_LOG2E * jax.lax.rsqrt(k2 + 1e-30)))
    kh = kbs.astype(jnp.bfloat16)
    kl = (kbs - kh.astype(jnp.float32)).astype(jnp.bfloat16)
    a = jnp.concatenate([kh, kh, kl], axis=1)

    eacc = None
    for base in range(0, N, _C):
        e_c = jnp.exp2(jax.lax.dot_general(
            a, mhl_ref[:, base:base + _C], (((1,), (0,)), ((), ())),
            preferred_element_type=jnp.float32))
        ep_ref[:, H + base:H + base + _C] = e_c
        if base == 0:
            ep_ref[:, N + H:N + 2 * H] = e_c[:, 0:H]
        if base == N - _C:
            ep_ref[:, 0:H] = e_c[:, _C - H:_C]
        part = e_c[:, 0:H]
        for col in range(H, _C, H):
            part = part + e_c[:, col:col + H]
        eacc = part if eacc is None else eacc + part
    esum = jnp.sum(eacc, axis=1, keepdims=True)

    ga = g / esum
    gb = 1.0 - g
    s1 = s[:, 1:2]
    r0 = s[:, 0:1] / s1
    r2 = s[:, 2:3] / s1
    eps = jnp.exp2(gamma * (-jnp.log2(s1)) + _LOG2_EPS)

    acc = None
    for base in range(0, N, _C):
        eext = ep_ref[:, base:base + _C + 2 * H]
        if base == 0:
            wpext = jnp.concatenate(
                [wp_ref[:, N - H:N], wp_ref[:, 0:_C + H]], axis=1)
        elif base == N - _C:
            wpext = jnp.concatenate(
                [wp_ref[:, base - H:N], wp_ref[:, 0:H]], axis=1)
        else:
            wpext = wp_ref[:, base - H:base + _C + H]
        wgext = eext * ga + wpext * gb
        wl = wgext[:, H - 1:_C + H - 1]
        wr = wgext[:, H + 1:_C + H + 1]
        wh = wgext[:, H:_C + H] + wl * r0 + wr * r2
        wc = jnp.exp2(gamma * jnp.log2(wh))
        out_ref[:, base:base + _C] = wc
        part = wc[:, 0:H]
        for col in range(H, _C, H):
            part = part + wc[:, col:col + H]
        acc = part if acc is None else acc + part

    total = jnp.sum(acc, axis=1, keepdims=True) + eps
    out_ref[...] = out_ref[...] * (1.0 / total)


@jax.jit
def kernel(k, beta, g, s, gamma, w_prev, memory):
    B, M = k.shape
    N = memory.shape[0]
    BB = 64
    memT = memory.T

    grid = (B // BB,)
    return pl.pallas_call(
        _ntm_block,
        grid=grid,
        in_specs=[
            pl.BlockSpec((BB, M), lambda i: (i, 0)),
            pl.BlockSpec((BB, 1), lambda i: (i, 0)),
            pl.BlockSpec((BB, 1), lambda i: (i, 0)),
            pl.BlockSpec((BB, 3), lambda i: (i, 0)),
            pl.BlockSpec((BB, 1), lambda i: (i, 0)),
            pl.BlockSpec((BB, N), lambda i: (i, 0)),
            pl.BlockSpec((M, N), lambda i: (0, 0)),
        ],
        out_specs=pl.BlockSpec((BB, N), lambda i: (i, 0)),
        out_shape=jax.ShapeDtypeStruct((B, N), jnp.float32),
        scratch_shapes=[
            pltpu.VMEM((3 * M, N), jnp.bfloat16),
            pltpu.VMEM((BB, N + 2 * _LANE), jnp.float32),
        ],
    )(k, beta, g, s, gamma, w_prev, memT)
